# Initial kernel scaffold; baseline (speedup 1.0000x reference)
import functools

import jax
import jax.numpy as jnp
from jax import lax
from jax.experimental import pallas as pl
from jax.experimental.pallas import tpu as pltpu
from jax.experimental.pallas import tpu_sc as plsc

B, L, D = 4096, 200, 128
LANES = 16


def kernel(inputs, pos_embedding):
    info = plsc.get_sparse_core_info()
    nc, ns = info.num_cores, info.num_subcores
    nw = nc * ns
    rows = B // nw

    mesh = plsc.VectorSubcoreMesh(core_axis_name="c", subcore_axis_name="s")

    @functools.partial(
        pl.kernel,
        mesh=mesh,
        out_type=jax.ShapeDtypeStruct((B, L, D), jnp.float32),
        scratch_types=[
            pltpu.VMEM((L, D), jnp.float32),
            pltpu.VMEM((L, D), jnp.float32),
            pltpu.VMEM((L, D), jnp.float32),
            pltpu.SemaphoreType.DMA,
            pltpu.SemaphoreType.DMA,
        ],
    )
    def sc_add(in_hbm, pe_hbm, out_hbm, pe_v, buf0, buf1, s0, s1):
        c = lax.axis_index("c")
        s = lax.axis_index("s")
        wid = s * nc + c
        base = wid * rows

        pltpu.sync_copy(pe_hbm.at[pl.ds(1, L)], pe_v)

        def add_pe(buf):
            def body(l, carry):
                for j in range(D // LANES):
                    sl = pl.ds(j * LANES, LANES)
                    buf[l, sl] = buf[l, sl] + pe_v[l, sl]
                return carry
            lax.fori_loop(0, L, body, 0)

        pltpu.async_copy(in_hbm.at[base], buf0, s0)
        pltpu.async_copy(in_hbm.at[base + 1], buf1, s1)

        def step(t, carry):
            r0 = base + 2 * t
            r1 = r0 + 1
            pltpu.make_async_copy(in_hbm.at[r0], buf0, s0).wait()
            add_pe(buf0)
            pltpu.async_copy(buf0, out_hbm.at[r0], s0)
            pltpu.make_async_copy(in_hbm.at[r1], buf1, s1).wait()
            add_pe(buf1)
            pltpu.async_copy(buf1, out_hbm.at[r1], s1)
            pltpu.make_async_copy(buf0, out_hbm.at[r0], s0).wait()
            pltpu.async_copy(in_hbm.at[r0 + 2], buf0, s0)
            pltpu.make_async_copy(buf1, out_hbm.at[r1], s1).wait()
            pltpu.async_copy(in_hbm.at[r1 + 2], buf1, s1)
            return carry

        lax.fori_loop(0, rows // 2 - 1, step, 0)

        r0 = base + rows - 2
        r1 = base + rows - 1
        pltpu.make_async_copy(in_hbm.at[r0], buf0, s0).wait()
        add_pe(buf0)
        pltpu.async_copy(buf0, out_hbm.at[r0], s0)
        pltpu.make_async_copy(in_hbm.at[r1], buf1, s1).wait()
        add_pe(buf1)
        pltpu.async_copy(buf1, out_hbm.at[r1], s1)
        pltpu.make_async_copy(buf0, out_hbm.at[r0], s0).wait()
        pltpu.make_async_copy(buf1, out_hbm.at[r1], s1).wait()

    return sc_add(inputs, pos_embedding)


# speedup vs baseline: 7.7767x; 7.7767x over previous
import functools

import jax
import jax.numpy as jnp
from jax import lax
from jax.experimental import pallas as pl
from jax.experimental.pallas import tpu as pltpu
from jax.experimental.pallas import tpu_sc as plsc

B, L, D = 4096, 200, 128
LANES = 16


def kernel(inputs, pos_embedding):
    info = plsc.get_sparse_core_info()
    nc, ns = info.num_cores, info.num_subcores
    nw = nc * ns
    rows = B // nw

    mesh = plsc.VectorSubcoreMesh(core_axis_name="c", subcore_axis_name="s")

    @functools.partial(
        pl.kernel,
        mesh=mesh,
        out_type=jax.ShapeDtypeStruct((B, L, D), jnp.float32),
        scratch_types=[
            pltpu.VMEM((208, D), jnp.float32),
            pltpu.VMEM((L, D), jnp.float32),
            pltpu.VMEM((L, D), jnp.float32),
            pltpu.SemaphoreType.DMA,
            pltpu.SemaphoreType.DMA,
        ],
    )
    def sc_add(in_hbm, pe_hbm, out_hbm, pe_v, buf0, buf1, s0, s1):
        c = lax.axis_index("c")
        s = lax.axis_index("s")
        wid = s * nc + c
        base = wid * rows

        pltpu.sync_copy(pe_hbm.at[pl.ds(0, 208)], pe_v)

        def add_pe(buf):
            def body(l, carry):
                for j in range(D // LANES):
                    sl = pl.ds(j * LANES, LANES)
                    buf[l, sl] = buf[l, sl] + pe_v[l + 1, sl]
                return carry
            lax.fori_loop(0, L, body, 0)

        pltpu.async_copy(in_hbm.at[base], buf0, s0)
        pltpu.async_copy(in_hbm.at[base + 1], buf1, s1)

        def step(t, carry):
            r0 = base + 2 * t
            r1 = r0 + 1
            pltpu.make_async_copy(in_hbm.at[r0], buf0, s0).wait()
            add_pe(buf0)
            pltpu.async_copy(buf0, out_hbm.at[r0], s0)
            pltpu.make_async_copy(in_hbm.at[r1], buf1, s1).wait()
            add_pe(buf1)
            pltpu.async_copy(buf1, out_hbm.at[r1], s1)
            pltpu.make_async_copy(buf0, out_hbm.at[r0], s0).wait()
            pltpu.async_copy(in_hbm.at[r0 + 2], buf0, s0)
            pltpu.make_async_copy(buf1, out_hbm.at[r1], s1).wait()
            pltpu.async_copy(in_hbm.at[r1 + 2], buf1, s1)
            return carry

        lax.fori_loop(0, rows // 2 - 1, step, 0)

        r0 = base + rows - 2
        r1 = base + rows - 1
        pltpu.make_async_copy(in_hbm.at[r0], buf0, s0).wait()
        add_pe(buf0)
        pltpu.async_copy(buf0, out_hbm.at[r0], s0)
        pltpu.make_async_copy(in_hbm.at[r1], buf1, s1).wait()
        add_pe(buf1)
        pltpu.async_copy(buf1, out_hbm.at[r1], s1)
        pltpu.make_async_copy(buf0, out_hbm.at[r0], s0).wait()
        pltpu.make_async_copy(buf1, out_hbm.at[r1], s1).wait()

    return sc_add(inputs, pos_embedding)


# speedup vs baseline: 9.4223x; 1.2116x over previous
import functools

import jax
import jax.numpy as jnp
from jax import lax
from jax.experimental import pallas as pl
from jax.experimental.pallas import tpu as pltpu
from jax.experimental.pallas import tpu_sc as plsc

B, L, D = 4096, 200, 128
LANES = 16
PE_ROWS = 208
NB = 2


def kernel(inputs, pos_embedding):
    info = plsc.get_sparse_core_info()
    nc, ns = info.num_cores, info.num_subcores
    nw = nc * ns
    rows = B // nw
    pairs = rows // NB

    mesh = plsc.VectorSubcoreMesh(core_axis_name="c", subcore_axis_name="s")

    @functools.partial(
        pl.kernel,
        mesh=mesh,
        out_type=jax.ShapeDtypeStruct((B, L, D), jnp.float32),
        scratch_types=[
            pltpu.VMEM((PE_ROWS, D), jnp.float32),
            pltpu.VMEM((NB, L, D), jnp.float32),
            pltpu.VMEM((NB, L, D), jnp.float32),
            pltpu.SemaphoreType.DMA,
            pltpu.SemaphoreType.DMA,
        ],
    )
    def sc_add(in_hbm, pe_hbm, out_hbm, pe_v, buf0, buf1, s0, s1):
        c = lax.axis_index("c")
        s = lax.axis_index("s")
        wid = s * nc + c
        base = wid * rows

        pltpu.sync_copy(pe_hbm.at[pl.ds(0, PE_ROWS)], pe_v)

        def add_pe(buf):
            def body(l, carry):
                for j in range(D // LANES):
                    sl = pl.ds(j * LANES, LANES)
                    pe = pe_v[l + 1, sl]
                    buf[0, l, sl] = buf[0, l, sl] + pe
                    buf[1, l, sl] = buf[1, l, sl] + pe
                return carry
            lax.fori_loop(0, L, body, 0)

        pltpu.async_copy(in_hbm.at[pl.ds(base, NB)], buf0, s0)
        pltpu.async_copy(in_hbm.at[pl.ds(base + NB, NB)], buf1, s1)

        def step(t, carry):
            r0 = base + 2 * NB * t
            r1 = r0 + NB
            pltpu.make_async_copy(in_hbm.at[pl.ds(r0, NB)], buf0, s0).wait()
            add_pe(buf0)
            pltpu.async_copy(buf0, out_hbm.at[pl.ds(r0, NB)], s0)
            pltpu.make_async_copy(in_hbm.at[pl.ds(r1, NB)], buf1, s1).wait()
            add_pe(buf1)
            pltpu.async_copy(buf1, out_hbm.at[pl.ds(r1, NB)], s1)
            pltpu.make_async_copy(buf0, out_hbm.at[pl.ds(r0, NB)], s0).wait()
            pltpu.async_copy(in_hbm.at[pl.ds(r0 + 2 * NB, NB)], buf0, s0)
            pltpu.make_async_copy(buf1, out_hbm.at[pl.ds(r1, NB)], s1).wait()
            pltpu.async_copy(in_hbm.at[pl.ds(r1 + 2 * NB, NB)], buf1, s1)
            return carry

        lax.fori_loop(0, pairs // 2 - 1, step, 0)

        r0 = base + rows - 2 * NB
        r1 = base + rows - NB
        pltpu.make_async_copy(in_hbm.at[pl.ds(r0, NB)], buf0, s0).wait()
        add_pe(buf0)
        pltpu.async_copy(buf0, out_hbm.at[pl.ds(r0, NB)], s0)
        pltpu.make_async_copy(in_hbm.at[pl.ds(r1, NB)], buf1, s1).wait()
        add_pe(buf1)
        pltpu.async_copy(buf1, out_hbm.at[pl.ds(r1, NB)], s1)
        pltpu.make_async_copy(buf0, out_hbm.at[pl.ds(r0, NB)], s0).wait()
        pltpu.make_async_copy(buf1, out_hbm.at[pl.ds(r1, NB)], s1).wait()

    return sc_add(inputs, pos_embedding)
